# Initial kernel scaffold; baseline (speedup 1.0000x reference)
import jax
import jax.numpy as jnp
from jax.experimental import pallas as pl


def kernel(x, edge_index, edge_attr, pickable, params):
    raise NotImplementedError("write your pallas kernel here")


# speedup vs baseline: 9.3324x; 9.3324x over previous
import functools

import jax
import jax.numpy as jnp
from jax import lax
from jax.experimental import pallas as pl
from jax.experimental.pallas import tpu as pltpu
from jax.experimental.pallas import tpu_sc as plsc

_N = 10000
_E = 320000
_DE = 16
_HID = 128
_NCLS = 10
_NPICK = 5000

_NCORE = 2
_NSUB = 16
_NPAD = 10240
_NW = _NCORE * _NSUB
_EPW = _E // _NW
_CH = 80
_NCHUNK = _EPW // _CH
_RPT = _NPAD // _NSUB

_PICK_PAD = 5120
_PPW = _PICK_PAD // _NW

_HIGH = jax.lax.Precision.HIGHEST

_sc_mesh = plsc.VectorSubcoreMesh(core_axis_name="c", subcore_axis_name="s")


def _emlp_body(ea_ref, e1w, e1b, e2w, e2b, m1, b1, m2, b2, out_ref):
    ea = ea_ref[...]
    ea = jnp.dot(ea, e1w[...], preferred_element_type=jnp.float32) + e1b[...]
    ea = jnp.dot(ea, e2w[...], preferred_element_type=jnp.float32) + e2b[...]
    h = jnp.maximum(
        jnp.dot(ea, m1[...], preferred_element_type=jnp.float32) + b1[...], 0.0)
    z = jnp.dot(h, m2[...], preferred_element_type=jnp.float32) + b2[...]
    out_ref[...] = jax.nn.sigmoid(z)


def _edge_mlp(edge_attr, e1w, e1b, e2w, e2b, m1, b1, m2, b2):
    be = 2000
    grid = _E // be
    full = lambda r, c: pl.BlockSpec((r, c), lambda i: (0, 0))
    return pl.pallas_call(
        _emlp_body,
        grid=(grid,),
        in_specs=[
            pl.BlockSpec((be, _DE), lambda i: (i, 0)),
            full(_DE, _DE), full(1, _DE), full(_DE, _DE), full(1, _DE),
            full(_DE, 4 * _HID), full(1, 4 * _HID),
            full(4 * _HID, _DE), full(1, _DE),
        ],
        out_specs=pl.BlockSpec((be, _DE), lambda i: (i, 0)),
        out_shape=jax.ShapeDtypeStruct((_E, _DE), jnp.float32),
    )(edge_attr, e1w, e1b, e2w, e2b, m1, b1, m2, b2)


def _pre0_body(x_ref, pa_ref, pb_ref, w_ref, dinv_ref, xs_ref):
    deg = pa_ref[...] + pb_ref[...] + 1.0
    dv = lax.rsqrt(deg)
    dinv_ref[...] = dv
    xw = jnp.dot(x_ref[...], w_ref[...], precision=_HIGH,
                 preferred_element_type=jnp.float32)
    xs_ref[...] = dv[:, 0:1] * xw


def _pre0(x, pa, pb, w0):
    bn = 1000
    return pl.pallas_call(
        _pre0_body,
        grid=(_N // bn,),
        in_specs=[
            pl.BlockSpec((bn, _HID), lambda i: (i, 0)),
            pl.BlockSpec((bn, _HID), lambda i: (i, 0)),
            pl.BlockSpec((bn, _HID), lambda i: (i, 0)),
            pl.BlockSpec((_HID, _HID), lambda i: (0, 0)),
        ],
        out_specs=[
            pl.BlockSpec((bn, _HID), lambda i: (i, 0)),
            pl.BlockSpec((bn, _HID), lambda i: (i, 0)),
        ],
        out_shape=[
            jax.ShapeDtypeStruct((_N, _HID), jnp.float32),
            jax.ShapeDtypeStruct((_N, _HID), jnp.float32),
        ],
    )(x, pa, pb, w0)


def _mid_body(aa_ref, ab_ref, xs_ref, dinv_ref, b_ref, w_ref, out_ref, *, j):
    dvj = dinv_ref[:, j:j + 1]
    dvn = dinv_ref[:, j + 1:j + 2]
    h = dvj * (aa_ref[...] + ab_ref[...] + xs_ref[...]) + b_ref[...]
    h = jnp.maximum(h, 0.0)
    out_ref[...] = dvn * jnp.dot(h, w_ref[...], precision=_HIGH,
                                 preferred_element_type=jnp.float32)


def _mid(aa, ab, xs, dinv, b, w, j):
    bn = 1000
    return pl.pallas_call(
        functools.partial(_mid_body, j=j),
        grid=(_N // bn,),
        in_specs=[
            pl.BlockSpec((bn, _HID), lambda i: (i, 0)),
            pl.BlockSpec((bn, _HID), lambda i: (i, 0)),
            pl.BlockSpec((bn, _HID), lambda i: (i, 0)),
            pl.BlockSpec((bn, _HID), lambda i: (i, 0)),
            pl.BlockSpec((1, _HID), lambda i: (0, 0)),
            pl.BlockSpec((_HID, _HID), lambda i: (0, 0)),
        ],
        out_specs=pl.BlockSpec((bn, _HID), lambda i: (i, 0)),
        out_shape=jax.ShapeDtypeStruct((_N, _HID), jnp.float32),
    )(aa, ab, xs, dinv, b, w)


def _final_body(aa_ref, ab_ref, xs_ref, dinv_ref, b_ref, w_ref, lb_ref, out_ref):
    dvj = dinv_ref[:, 3:4]
    h = dvj * (aa_ref[...] + ab_ref[...] + xs_ref[...]) + b_ref[...]
    logits = jnp.dot(h, w_ref[...], precision=_HIGH,
                     preferred_element_type=jnp.float32) + lb_ref[...]
    m = jnp.max(logits, axis=1, keepdims=True)
    p = jnp.exp(logits - m)
    out_ref[...] = p / jnp.sum(p, axis=1, keepdims=True)


def _final(aa, ab, xs, dinv, b, w_pad, lb_pad):
    bn = 1000
    return pl.pallas_call(
        _final_body,
        grid=(_N // bn,),
        in_specs=[
            pl.BlockSpec((bn, _HID), lambda i: (i, 0)),
            pl.BlockSpec((bn, _HID), lambda i: (i, 0)),
            pl.BlockSpec((bn, _HID), lambda i: (i, 0)),
            pl.BlockSpec((bn, _HID), lambda i: (i, 0)),
            pl.BlockSpec((1, _HID), lambda i: (0, 0)),
            pl.BlockSpec((_HID, _HID), lambda i: (0, 0)),
            pl.BlockSpec((1, _HID), lambda i: (0, 0)),
        ],
        out_specs=pl.BlockSpec((bn, _HID), lambda i: (i, 0)),
        out_shape=jax.ShapeDtypeStruct((_N, _HID), jnp.float32),
    )(aa, ab, xs, dinv, b, w_pad, lb_pad)


@functools.partial(
    pl.kernel,
    out_type=jax.ShapeDtypeStruct((_NCORE, _NPAD, _HID), jnp.float32),
    mesh=_sc_mesh,
    scratch_types=[
        pltpu.VMEM_SHARED((_NPAD, _HID), jnp.float32),
        pltpu.VMEM((_CH,), jnp.int32),
        pltpu.VMEM((_CH, _DE), jnp.float32),
        pltpu.VMEM((_CH, _HID), jnp.float32),
    ],
)
def _deg_kernel(col_hbm, w4_hbm, z_hbm, out_hbm, acc, coli, wbuf, pay):
    c = lax.axis_index("c")
    s = lax.axis_index("s")
    wid = c * _NSUB + s
    pltpu.sync_copy(z_hbm.at[pl.ds(s * _RPT, _RPT)],
                    acc.at[pl.ds(s * _RPT, _RPT)])
    pltpu.sync_copy(z_hbm.at[pl.ds(0, _CH)], pay)
    plsc.subcore_barrier()

    def body(i, carry):
        b = wid * _EPW + i * _CH
        pltpu.sync_copy(col_hbm.at[pl.ds(b, _CH)], coli)
        pltpu.sync_copy(w4_hbm.at[pl.ds(b, _CH)], wbuf)

        def cp(e8, carry2):
            for u in range(8):
                e = e8 * 8 + u
                pay[e, pl.ds(0, _DE)] = wbuf[e, :]
            return carry2

        lax.fori_loop(0, _CH // 8, cp, 0)
        pltpu.sync_copy(pay, acc.at[coli], add=True)
        return carry

    lax.fori_loop(0, _NCHUNK, body, 0)
    plsc.subcore_barrier()
    pltpu.sync_copy(acc.at[pl.ds(s * _RPT, _RPT)],
                    out_hbm.at[c, pl.ds(s * _RPT, _RPT)])


def _make_prop(j):
    @functools.partial(
        pl.kernel,
        out_type=jax.ShapeDtypeStruct((_NCORE, _NPAD, _HID), jnp.float32),
        mesh=_sc_mesh,
        scratch_types=[
            pltpu.VMEM_SHARED((_NPAD, _HID), jnp.float32),
            pltpu.VMEM((_CH,), jnp.int32),
            pltpu.VMEM((_CH,), jnp.int32),
            pltpu.VMEM((_CH, _DE), jnp.float32),
            pltpu.VMEM((_CH, _HID), jnp.float32),
            pltpu.SemaphoreType.DMA,
        ],
    )
    def _prop(xs_hbm, row_hbm, col_hbm, w4_hbm, z_hbm, out_hbm,
              acc, rowi, coli, wbuf, rows, sem):
        c = lax.axis_index("c")
        s = lax.axis_index("s")
        wid = c * _NSUB + s
        pltpu.sync_copy(z_hbm.at[pl.ds(s * _RPT, _RPT)],
                        acc.at[pl.ds(s * _RPT, _RPT)])
        plsc.subcore_barrier()

        def body(i, carry):
            b = wid * _EPW + i * _CH
            pltpu.sync_copy(row_hbm.at[pl.ds(b, _CH)], rowi)
            pltpu.sync_copy(col_hbm.at[pl.ds(b, _CH)], coli)
            pltpu.sync_copy(w4_hbm.at[pl.ds(b, _CH)], wbuf)
            pltpu.async_copy(xs_hbm.at[rowi], rows, sem).wait()

            def scale(e, carry2):
                w = wbuf[e, :][j]
                for g in range(_HID // 16):
                    rows[e, pl.ds(g * 16, 16)] = rows[e, pl.ds(g * 16, 16)] * w
                return carry2

            lax.fori_loop(0, _CH, scale, 0)
            pltpu.sync_copy(rows, acc.at[coli], add=True)
            return carry

        lax.fori_loop(0, _NCHUNK, body, 0)
        plsc.subcore_barrier()
        pltpu.sync_copy(acc.at[pl.ds(s * _RPT, _RPT)],
                        out_hbm.at[c, pl.ds(s * _RPT, _RPT)])

    return _prop


_prop_kernels = [_make_prop(j) for j in range(4)]


@functools.partial(
    pl.kernel,
    out_type=jax.ShapeDtypeStruct((_PICK_PAD, _HID), jnp.float32),
    mesh=_sc_mesh,
    scratch_types=[
        pltpu.VMEM((_PPW,), jnp.int32),
        pltpu.VMEM((_PPW, _HID), jnp.float32),
        pltpu.SemaphoreType.DMA,
    ],
)
def _pick_kernel(probs_hbm, pick_hbm, out_hbm, idxv, rowsv, sem):
    c = lax.axis_index("c")
    s = lax.axis_index("s")
    wid = c * _NSUB + s
    base = wid * _PPW
    pltpu.sync_copy(pick_hbm.at[pl.ds(base, _PPW)], idxv)
    for h in range(_PPW // _CH):
        pltpu.async_copy(probs_hbm.at[idxv.at[pl.ds(h * _CH, _CH)]],
                         rowsv.at[pl.ds(h * _CH, _CH)], sem).wait()
    pltpu.sync_copy(rowsv, out_hbm.at[pl.ds(base, _PPW)])


def kernel(x, edge_index, edge_attr, pickable, params):
    p = params
    row = edge_index[0].astype(jnp.int32)
    col = edge_index[1].astype(jnp.int32)
    pick = jnp.concatenate(
        [pickable.astype(jnp.int32),
         jnp.zeros((_PICK_PAD - _NPICK,), jnp.int32)])

    m1 = jnp.concatenate([p['c%d_m1_w' % j] for j in range(4)], axis=1)
    b1 = jnp.concatenate([p['c%d_m1_b' % j] for j in range(4)])[None, :]
    m2 = jnp.zeros((4 * _HID, _DE), jnp.float32)
    for j in range(4):
        m2 = m2.at[j * _HID:(j + 1) * _HID, j].set(p['c%d_m2_w' % j][:, 0])
    b2 = jnp.full((_DE,), -1e30, jnp.float32)
    b2 = b2.at[:4].set(jnp.stack([p['c%d_m2_b' % j][0] for j in range(4)]))
    b2 = b2[None, :]

    lin_w_pad = jnp.zeros((_HID, _HID), jnp.float32)
    lin_w_pad = lin_w_pad.at[:, :_NCLS].set(p['lin_w'])
    lin_b_pad = jnp.full((_HID,), -1e30, jnp.float32)
    lin_b_pad = lin_b_pad.at[:_NCLS].set(p['lin_b'])
    lin_b_pad = lin_b_pad[None, :]

    w4 = _edge_mlp(edge_attr,
                   p['enc1_w'], p['enc1_b'][None, :],
                   p['enc2_w'], p['enc2_b'][None, :],
                   m1, b1, m2, b2)

    z128 = jnp.zeros((_NPAD, _HID), jnp.float32)

    degp = _deg_kernel(col, w4, z128)
    dinv, xs = _pre0(x, degp[0], degp[1], p['c0_lin_w'])

    for j in range(4):
        accs = _prop_kernels[j](xs, row, col, w4, z128)
        if j < 3:
            xs = _mid(accs[0], accs[1], xs, dinv,
                      p['c%d_bias' % j][None, :], p['c%d_lin_w' % (j + 1)], j)
        else:
            probs = _final(accs[0], accs[1], xs, dinv,
                           p['c3_bias'][None, :], lin_w_pad, lin_b_pad)

    picked = _pick_kernel(probs, pick)
    return picked[:_NPICK, :_NCLS]


# speedup vs baseline: 18.7613x; 2.0103x over previous
import functools

import jax
import jax.numpy as jnp
from jax import lax
from jax.experimental import pallas as pl
from jax.experimental.pallas import tpu as pltpu
from jax.experimental.pallas import tpu_sc as plsc

_N = 10000
_E = 320000
_DE = 16
_HID = 128
_NCLS = 10
_NPICK = 5000

_NCORE = 2
_NSUB = 16
_NPAD = 10240
_NW = _NCORE * _NSUB
_EPW = _E // _NW
_CH = 80
_NCHUNK = _EPW // _CH
_RPT = _NPAD // _NSUB

_PICK_PAD = 5120
_PPW = _PICK_PAD // _NW

_HIGH = jax.lax.Precision.HIGHEST

_sc_mesh = plsc.VectorSubcoreMesh(core_axis_name="c", subcore_axis_name="s")


def _emlp_body(ea_ref, e1w, e1b, e2w, e2b, m1, b1, m2, b2, b2t, out_ref, out_t_ref):
    ea = ea_ref[...]
    ea = jnp.dot(ea, e1w[...], preferred_element_type=jnp.float32) + e1b[...]
    ea = jnp.dot(ea, e2w[...], preferred_element_type=jnp.float32) + e2b[...]
    h = jnp.maximum(
        jnp.dot(ea, m1[...], preferred_element_type=jnp.float32) + b1[...], 0.0)
    z = jnp.dot(h, m2[...], preferred_element_type=jnp.float32) + b2[...]
    out_ref[...] = jax.nn.sigmoid(z)
    zt = lax.dot_general(m2[...], h, (((0,), (1,)), ((), ())),
                         preferred_element_type=jnp.float32) + b2t[...]
    out_t_ref[...] = jax.nn.sigmoid(zt)


def _edge_mlp(edge_attr, e1w, e1b, e2w, e2b, m1, b1, m2, b2):
    be = 2560
    grid = _E // be
    full = lambda r, c: pl.BlockSpec((r, c), lambda i: (0, 0))
    return pl.pallas_call(
        _emlp_body,
        grid=(grid,),
        in_specs=[
            pl.BlockSpec((be, _DE), lambda i: (i, 0)),
            full(_DE, _DE), full(1, _DE), full(_DE, _DE), full(1, _DE),
            full(_DE, 4 * _HID), full(1, 4 * _HID),
            full(4 * _HID, _DE), full(1, _DE), full(_DE, 1),
        ],
        out_specs=[
            pl.BlockSpec((be, _DE), lambda i: (i, 0)),
            pl.BlockSpec((_DE, be), lambda i: (0, i)),
        ],
        out_shape=[
            jax.ShapeDtypeStruct((_E, _DE), jnp.float32),
            jax.ShapeDtypeStruct((_DE, _E), jnp.float32),
        ],
    )(edge_attr, e1w, e1b, e2w, e2b, m1, b1, m2, b2, b2.reshape(_DE, 1))


def _pre0_body(x_ref, pa_ref, pb_ref, w_ref, dinv_ref, xs_ref):
    deg = pa_ref[...] + pb_ref[...] + 1.0
    dv = lax.rsqrt(deg)
    dinv_ref[...] = dv
    xw = jnp.dot(x_ref[...], w_ref[...], precision=_HIGH,
                 preferred_element_type=jnp.float32)
    xs_ref[...] = dv[:, 0:1] * xw


def _pre0(x, pa, pb, w0):
    bn = 1000
    return pl.pallas_call(
        _pre0_body,
        grid=(_N // bn,),
        in_specs=[
            pl.BlockSpec((bn, _HID), lambda i: (i, 0)),
            pl.BlockSpec((bn, _HID), lambda i: (i, 0)),
            pl.BlockSpec((bn, _HID), lambda i: (i, 0)),
            pl.BlockSpec((_HID, _HID), lambda i: (0, 0)),
        ],
        out_specs=[
            pl.BlockSpec((bn, _HID), lambda i: (i, 0)),
            pl.BlockSpec((bn, _HID), lambda i: (i, 0)),
        ],
        out_shape=[
            jax.ShapeDtypeStruct((_N, _HID), jnp.float32),
            jax.ShapeDtypeStruct((_N, _HID), jnp.float32),
        ],
    )(x, pa, pb, w0)


def _mid_body(aa_ref, ab_ref, xs_ref, dinv_ref, b_ref, w_ref, out_ref, *, j):
    dvj = dinv_ref[:, j:j + 1]
    dvn = dinv_ref[:, j + 1:j + 2]
    h = dvj * (aa_ref[...] + ab_ref[...] + xs_ref[...]) + b_ref[...]
    h = jnp.maximum(h, 0.0)
    out_ref[...] = dvn * jnp.dot(h, w_ref[...], precision=_HIGH,
                                 preferred_element_type=jnp.float32)


def _mid(aa, ab, xs, dinv, b, w, j):
    bn = 1000
    return pl.pallas_call(
        functools.partial(_mid_body, j=j),
        grid=(_N // bn,),
        in_specs=[
            pl.BlockSpec((bn, _HID), lambda i: (i, 0)),
            pl.BlockSpec((bn, _HID), lambda i: (i, 0)),
            pl.BlockSpec((bn, _HID), lambda i: (i, 0)),
            pl.BlockSpec((bn, _HID), lambda i: (i, 0)),
            pl.BlockSpec((1, _HID), lambda i: (0, 0)),
            pl.BlockSpec((_HID, _HID), lambda i: (0, 0)),
        ],
        out_specs=pl.BlockSpec((bn, _HID), lambda i: (i, 0)),
        out_shape=jax.ShapeDtypeStruct((_N, _HID), jnp.float32),
    )(aa, ab, xs, dinv, b, w)


def _final_body(aa_ref, ab_ref, xs_ref, dinv_ref, b_ref, w_ref, lb_ref, out_ref):
    dvj = dinv_ref[:, 3:4]
    h = dvj * (aa_ref[...] + ab_ref[...] + xs_ref[...]) + b_ref[...]
    logits = jnp.dot(h, w_ref[...], precision=_HIGH,
                     preferred_element_type=jnp.float32) + lb_ref[...]
    m = jnp.max(logits, axis=1, keepdims=True)
    p = jnp.exp(logits - m)
    out_ref[...] = p / jnp.sum(p, axis=1, keepdims=True)


def _final(aa, ab, xs, dinv, b, w_pad, lb_pad):
    bn = 1000
    return pl.pallas_call(
        _final_body,
        grid=(_N // bn,),
        in_specs=[
            pl.BlockSpec((bn, _HID), lambda i: (i, 0)),
            pl.BlockSpec((bn, _HID), lambda i: (i, 0)),
            pl.BlockSpec((bn, _HID), lambda i: (i, 0)),
            pl.BlockSpec((bn, _HID), lambda i: (i, 0)),
            pl.BlockSpec((1, _HID), lambda i: (0, 0)),
            pl.BlockSpec((_HID, _HID), lambda i: (0, 0)),
            pl.BlockSpec((1, _HID), lambda i: (0, 0)),
        ],
        out_specs=pl.BlockSpec((bn, _HID), lambda i: (i, 0)),
        out_shape=jax.ShapeDtypeStruct((_N, _HID), jnp.float32),
    )(aa, ab, xs, dinv, b, w_pad, lb_pad)


@functools.partial(
    pl.kernel,
    out_type=jax.ShapeDtypeStruct((_NCORE, _NPAD, _HID), jnp.float32),
    mesh=_sc_mesh,
    scratch_types=[
        pltpu.VMEM_SHARED((_NPAD, _HID), jnp.float32),
        pltpu.VMEM((_CH,), jnp.int32),
        pltpu.VMEM((_CH, _DE), jnp.float32),
        pltpu.VMEM((_CH, _HID), jnp.float32),
    ],
)
def _deg_kernel(col_hbm, w4_hbm, z_hbm, out_hbm, acc, coli, wbuf, pay):
    c = lax.axis_index("c")
    s = lax.axis_index("s")
    wid = c * _NSUB + s
    pltpu.sync_copy(z_hbm.at[pl.ds(s * _RPT, _RPT)],
                    acc.at[pl.ds(s * _RPT, _RPT)])
    pltpu.sync_copy(z_hbm.at[pl.ds(0, _CH)], pay)
    plsc.subcore_barrier()

    def body(i, carry):
        b = wid * _EPW + i * _CH
        pltpu.sync_copy(col_hbm.at[pl.ds(b, _CH)], coli)
        pltpu.sync_copy(w4_hbm.at[pl.ds(b, _CH)], wbuf)

        def cp(e8, carry2):
            for u in range(8):
                e = e8 * 8 + u
                pay[e, pl.ds(0, _DE)] = wbuf[e, :]
            return carry2

        lax.fori_loop(0, _CH // 8, cp, 0)
        pltpu.sync_copy(pay, acc.at[coli], add=True)
        return carry

    lax.fori_loop(0, _NCHUNK, body, 0)
    plsc.subcore_barrier()
    pltpu.sync_copy(acc.at[pl.ds(s * _RPT, _RPT)],
                    out_hbm.at[c, pl.ds(s * _RPT, _RPT)])


_NBUF = 3
_IDXB = 6
_MAIN = 120


def _make_prop():
    @functools.partial(
        pl.kernel,
        out_type=jax.ShapeDtypeStruct((_NCORE, _NPAD, _HID), jnp.float32),
        mesh=_sc_mesh,
        scratch_types=[
            pltpu.VMEM_SHARED((_NPAD, _HID), jnp.float32),
            pltpu.VMEM((_EPW,), jnp.float32),
            pltpu.VMEM((_NBUF * _CH, _HID), jnp.float32),
        ] + [pltpu.VMEM((_CH,), jnp.int32) for _ in range(2 * _IDXB)]
          + [pltpu.SemaphoreType.DMA for _ in range(2 * _NBUF + _IDXB)],
    )
    def _prop(xs_hbm, row_hbm, col_hbm, wj_hbm, z_hbm, out_hbm,
              acc, wj, bigbuf,
              r0, r1, r2, r3, r4, r5, cb0, cb1, cb2, cb3, cb4, cb5,
              g0, g1, g2, s0, s1, s2, i0, i1, i2, i3, i4, i5):
        rowb = (r0, r1, r2, r3, r4, r5)
        colb = (cb0, cb1, cb2, cb3, cb4, cb5)
        bufs = tuple(bigbuf.at[pl.ds(u * _CH, _CH)] for u in range(_NBUF))
        gsem = (g0, g1, g2)
        ssem = (s0, s1, s2)
        isem = (i0, i1, i2, i3, i4, i5)
        c = lax.axis_index("c")
        s = lax.axis_index("s")
        wid = c * _NSUB + s
        ebase = wid * _EPW
        pltpu.sync_copy(wj_hbm.at[pl.ds(ebase, _EPW)], wj)
        pltpu.sync_copy(z_hbm.at[pl.ds(s * _RPT, _RPT)],
                        acc.at[pl.ds(s * _RPT, _RPT)])
        plsc.subcore_barrier()

        def idx_descs(i, q):
            return (
                pltpu.make_async_copy(
                    row_hbm.at[pl.ds(ebase + i * _CH, _CH)], rowb[q], isem[q]),
                pltpu.make_async_copy(
                    col_hbm.at[pl.ds(ebase + i * _CH, _CH)], colb[q], isem[q]),
            )

        def fire_idx(i, q):
            for d in idx_descs(i, q):
                d.start()

        def wait_idx(i, q):
            for d in idx_descs(i, q):
                d.wait()

        def gather_desc(u, q):
            return pltpu.make_async_copy(xs_hbm.at[rowb[q]], bufs[u], gsem[u])

        def scatter_desc(u, q):
            return pltpu.make_async_copy(bufs[u], acc.at[colb[q]], ssem[u])

        def scale(i, u):
            buf = bufs[u]

            def do16(t16, carry):
                wv = wj[pl.ds(i * _CH + t16 * 16, 16)]
                e0 = t16 * 16
                for t in range(16):
                    sc = wv[t]
                    for g in range(_HID // 16):
                        buf[e0 + t, pl.ds(g * 16, 16)] = (
                            buf[e0 + t, pl.ds(g * 16, 16)] * sc)
                return carry

            lax.fori_loop(0, _CH // 16, do16, 0)

        def step(i, slot, fire_next_gather, fire_next_idx, guard_sw):
            u = slot % _NBUF
            v = (slot + 2) % _NBUF
            q2 = (slot + 2) % _IDXB
            q5 = (slot + 5) % _IDXB
            qp = (slot + 5) % _IDXB
            gather_desc(u, slot).wait()
            scale(i, u)
            if guard_sw:
                @pl.when(i >= 1)
                def _():
                    scatter_desc(v, qp).wait()
            else:
                scatter_desc(v, qp).wait()
            if fire_next_gather:
                wait_idx(i + 2, q2)
                gather_desc(v, q2).start()
            scatter_desc(u, slot).start(add=True)
            if fire_next_idx:
                fire_idx(i + 5, q5)

        for i in range(5):
            fire_idx(i, i)
        wait_idx(0, 0)
        gather_desc(0, 0).start()
        wait_idx(1, 1)
        gather_desc(1, 1).start()

        def outer(k, carry):
            i0_ = k * _IDXB
            for slot in range(_IDXB):
                step(i0_ + slot, slot, True, True, guard_sw=True)
            return carry

        lax.fori_loop(0, _MAIN // _IDXB, outer, 0)
        for i in range(_MAIN, _NCHUNK):
            step(i, i % _IDXB, fire_next_gather=(i + 2 < _NCHUNK),
                 fire_next_idx=False, guard_sw=False)
        scatter_desc((_NCHUNK - 1) % _NBUF, (_NCHUNK - 1) % _IDXB).wait()
        plsc.subcore_barrier()
        pltpu.sync_copy(acc.at[pl.ds(s * _RPT, _RPT)],
                        out_hbm.at[c, pl.ds(s * _RPT, _RPT)])

    return _prop


_prop_kernel = _make_prop()


@functools.partial(
    pl.kernel,
    out_type=jax.ShapeDtypeStruct((_PICK_PAD, _HID), jnp.float32),
    mesh=_sc_mesh,
    scratch_types=[
        pltpu.VMEM((_PPW,), jnp.int32),
        pltpu.VMEM((_PPW, _HID), jnp.float32),
        pltpu.SemaphoreType.DMA,
    ],
)
def _pick_kernel(probs_hbm, pick_hbm, out_hbm, idxv, rowsv, sem):
    c = lax.axis_index("c")
    s = lax.axis_index("s")
    wid = c * _NSUB + s
    base = wid * _PPW
    pltpu.sync_copy(pick_hbm.at[pl.ds(base, _PPW)], idxv)
    for h in range(_PPW // _CH):
        pltpu.async_copy(probs_hbm.at[idxv.at[pl.ds(h * _CH, _CH)]],
                         rowsv.at[pl.ds(h * _CH, _CH)], sem).wait()
    pltpu.sync_copy(rowsv, out_hbm.at[pl.ds(base, _PPW)])


def kernel(x, edge_index, edge_attr, pickable, params):
    p = params
    row = edge_index[0].astype(jnp.int32)
    col = edge_index[1].astype(jnp.int32)
    pick = jnp.concatenate(
        [pickable.astype(jnp.int32),
         jnp.zeros((_PICK_PAD - _NPICK,), jnp.int32)])

    m1 = jnp.concatenate([p['c%d_m1_w' % j] for j in range(4)], axis=1)
    b1 = jnp.concatenate([p['c%d_m1_b' % j] for j in range(4)])[None, :]
    m2 = jnp.zeros((4 * _HID, _DE), jnp.float32)
    for j in range(4):
        m2 = m2.at[j * _HID:(j + 1) * _HID, j].set(p['c%d_m2_w' % j][:, 0])
    b2 = jnp.full((_DE,), -1e30, jnp.float32)
    b2 = b2.at[:4].set(jnp.stack([p['c%d_m2_b' % j][0] for j in range(4)]))
    b2 = b2[None, :]

    lin_w_pad = jnp.zeros((_HID, _HID), jnp.float32)
    lin_w_pad = lin_w_pad.at[:, :_NCLS].set(p['lin_w'])
    lin_b_pad = jnp.full((_HID,), -1e30, jnp.float32)
    lin_b_pad = lin_b_pad.at[:_NCLS].set(p['lin_b'])
    lin_b_pad = lin_b_pad[None, :]

    w4, w4t = _edge_mlp(edge_attr,
                   p['enc1_w'], p['enc1_b'][None, :],
                   p['enc2_w'], p['enc2_b'][None, :],
                   m1, b1, m2, b2)

    z128 = jnp.zeros((_NPAD, _HID), jnp.float32)

    degp = _deg_kernel(col, w4, z128)
    dinv, xs = _pre0(x, degp[0], degp[1], p['c0_lin_w'])

    for j in range(4):
        accs = _prop_kernel(xs, row, col, w4t[j], z128)
        if j < 3:
            xs = _mid(accs[0], accs[1], xs, dinv,
                      p['c%d_bias' % j][None, :], p['c%d_lin_w' % (j + 1)], j)
        else:
            probs = _final(accs[0], accs[1], xs, dinv,
                           p['c3_bias'][None, :], lin_w_pad, lin_b_pad)

    picked = _pick_kernel(probs, pick)
    return picked[:_NPICK, :_NCLS]


# speedup vs baseline: 23.8713x; 1.2724x over previous
import functools

import jax
import jax.numpy as jnp
from jax import lax
from jax.experimental import pallas as pl
from jax.experimental.pallas import tpu as pltpu
from jax.experimental.pallas import tpu_sc as plsc

_N = 10000
_E = 320000
_DE = 16
_HID = 128
_NCLS = 10
_NPICK = 5000

_NCORE = 2
_NSUB = 16
_NPAD = 10240
_NW = _NCORE * _NSUB
_EPW = _E // _NW
_CH = 80
_NCHUNK = _EPW // _CH
_RPT = _NPAD // _NSUB

_PICK_PAD = 5120
_PPW = _PICK_PAD // _NW

_HIGH = jax.lax.Precision.HIGHEST

_sc_mesh = plsc.VectorSubcoreMesh(core_axis_name="c", subcore_axis_name="s")


def _emlp_body(eat_ref, m1c, b1c, m2, m2b, b2, b2t, out_ref, out_t_ref):
    eat = eat_ref[...].astype(jnp.bfloat16)
    ht = jnp.maximum(
        lax.dot_general(m1c[...], eat, (((0,), (0,)), ((), ())),
                        preferred_element_type=jnp.float32) + b1c[...], 0.0)
    htb = ht.astype(jnp.bfloat16)
    z = lax.dot_general(htb, m2b[...], (((0,), (0,)), ((), ())),
                        preferred_element_type=jnp.float32) + b2[...]
    out_ref[...] = jax.nn.sigmoid(z)
    zt = lax.dot_general(m2b[...], htb, (((0,), (0,)), ((), ())),
                         preferred_element_type=jnp.float32) + b2t[...]
    out_t_ref[...] = jax.nn.sigmoid(zt)


def _edge_mlp(eat, m1c, b1c, m2, b2):
    be = 2560
    grid = _E // be
    full = lambda r, c: pl.BlockSpec((r, c), lambda i: (0, 0))
    return pl.pallas_call(
        _emlp_body,
        grid=(grid,),
        in_specs=[
            pl.BlockSpec((_DE, be), lambda i: (0, i)),
            full(_DE, 4 * _HID), full(4 * _HID, 1),
            full(4 * _HID, _DE), full(4 * _HID, _DE), full(1, _DE),
            full(_DE, 1),
        ],
        out_specs=[
            pl.BlockSpec((be, _DE), lambda i: (i, 0)),
            pl.BlockSpec((_DE, be), lambda i: (0, i)),
        ],
        out_shape=[
            jax.ShapeDtypeStruct((_E, _DE), jnp.float32),
            jax.ShapeDtypeStruct((_DE, _E), jnp.float32),
        ],
    )(eat, m1c.astype(jnp.bfloat16), b1c.reshape(4 * _HID, 1),
      m2, m2.astype(jnp.bfloat16), b2, b2.reshape(_DE, 1))


def _pre0_body(x_ref, pa_ref, pb_ref, w_ref, dinv_ref, xs_ref):
    deg = pa_ref[...] + pb_ref[...] + 1.0
    dv = lax.rsqrt(deg)
    dinv_ref[...] = dv
    xw = jnp.dot(x_ref[...], w_ref[...], precision=_HIGH,
                 preferred_element_type=jnp.float32)
    xs_ref[...] = dv[:, 0:1] * xw


def _pre0(x, pa, pb, w0):
    bn = 1000
    return pl.pallas_call(
        _pre0_body,
        grid=(_N // bn,),
        in_specs=[
            pl.BlockSpec((bn, _HID), lambda i: (i, 0)),
            pl.BlockSpec((bn, _HID), lambda i: (i, 0)),
            pl.BlockSpec((bn, _HID), lambda i: (i, 0)),
            pl.BlockSpec((_HID, _HID), lambda i: (0, 0)),
        ],
        out_specs=[
            pl.BlockSpec((bn, _HID), lambda i: (i, 0)),
            pl.BlockSpec((bn, _HID), lambda i: (i, 0)),
        ],
        out_shape=[
            jax.ShapeDtypeStruct((_N, _HID), jnp.float32),
            jax.ShapeDtypeStruct((_N, _HID), jnp.float32),
        ],
    )(x, pa, pb, w0)


def _mid_body(aa_ref, ab_ref, xs_ref, dinv_ref, b_ref, w_ref, out_ref, *, j):
    dvj = dinv_ref[:, j:j + 1]
    dvn = dinv_ref[:, j + 1:j + 2]
    h = dvj * (aa_ref[...] + ab_ref[...] + xs_ref[...]) + b_ref[...]
    h = jnp.maximum(h, 0.0)
    out_ref[...] = dvn * jnp.dot(h, w_ref[...], precision=_HIGH,
                                 preferred_element_type=jnp.float32)


def _mid(aa, ab, xs, dinv, b, w, j):
    bn = 1000
    return pl.pallas_call(
        functools.partial(_mid_body, j=j),
        grid=(_N // bn,),
        in_specs=[
            pl.BlockSpec((bn, _HID), lambda i: (i, 0)),
            pl.BlockSpec((bn, _HID), lambda i: (i, 0)),
            pl.BlockSpec((bn, _HID), lambda i: (i, 0)),
            pl.BlockSpec((bn, _HID), lambda i: (i, 0)),
            pl.BlockSpec((1, _HID), lambda i: (0, 0)),
            pl.BlockSpec((_HID, _HID), lambda i: (0, 0)),
        ],
        out_specs=pl.BlockSpec((bn, _HID), lambda i: (i, 0)),
        out_shape=jax.ShapeDtypeStruct((_N, _HID), jnp.float32),
    )(aa, ab, xs, dinv, b, w)


def _final_body(aa_ref, ab_ref, xs_ref, dinv_ref, b_ref, w_ref, lb_ref, out_ref):
    dvj = dinv_ref[:, 3:4]
    h = dvj * (aa_ref[...] + ab_ref[...] + xs_ref[...]) + b_ref[...]
    logits = jnp.dot(h, w_ref[...], precision=_HIGH,
                     preferred_element_type=jnp.float32) + lb_ref[...]
    m = jnp.max(logits, axis=1, keepdims=True)
    p = jnp.exp(logits - m)
    out_ref[...] = p / jnp.sum(p, axis=1, keepdims=True)


def _final(aa, ab, xs, dinv, b, w_pad, lb_pad):
    bn = 1000
    return pl.pallas_call(
        _final_body,
        grid=(_N // bn,),
        in_specs=[
            pl.BlockSpec((bn, _HID), lambda i: (i, 0)),
            pl.BlockSpec((bn, _HID), lambda i: (i, 0)),
            pl.BlockSpec((bn, _HID), lambda i: (i, 0)),
            pl.BlockSpec((bn, _HID), lambda i: (i, 0)),
            pl.BlockSpec((1, _HID), lambda i: (0, 0)),
            pl.BlockSpec((_HID, _HID), lambda i: (0, 0)),
            pl.BlockSpec((1, _HID), lambda i: (0, 0)),
        ],
        out_specs=pl.BlockSpec((bn, _HID), lambda i: (i, 0)),
        out_shape=jax.ShapeDtypeStruct((_N, _HID), jnp.float32),
    )(aa, ab, xs, dinv, b, w_pad, lb_pad)


_DNB = 2
_DIX = 5
_DCH = 40
_DNCH = _EPW // _DCH


@functools.partial(
    pl.kernel,
    out_type=jax.ShapeDtypeStruct((_NCORE, _NPAD, _HID), jnp.float32),
    mesh=_sc_mesh,
    scratch_types=[
        pltpu.VMEM_SHARED((_NPAD, _HID), jnp.float32),
        pltpu.VMEM((_DNB * _DCH, _HID), jnp.float32),
    ] + [pltpu.VMEM((_DCH,), jnp.int32) for _ in range(_DIX)]
      + [pltpu.VMEM((_DCH, _DE), jnp.float32) for _ in range(_DIX)]
      + [pltpu.SemaphoreType.DMA for _ in range(_DNB + _DIX)],
)
def _deg_kernel(col_hbm, w4_hbm, z_hbm, out_hbm, acc, bigpay,
                cb0, cb1, cb2, cb3, cb4,
                wb0, wb1, wb2, wb3, wb4,
                s0, s1, i0, i1, i2, i3, i4):
    colb = (cb0, cb1, cb2, cb3, cb4)
    wbuf = (wb0, wb1, wb2, wb3, wb4)
    pays = tuple(bigpay.at[pl.ds(u * _DCH, _DCH)] for u in range(_DNB))
    ssem = (s0, s1)
    isem = (i0, i1, i2, i3, i4)
    c = lax.axis_index("c")
    s = lax.axis_index("s")
    wid = c * _NSUB + s
    ebase = wid * _EPW
    pltpu.sync_copy(z_hbm.at[pl.ds(s * _RPT, _RPT)],
                    acc.at[pl.ds(s * _RPT, _RPT)])
    pltpu.sync_copy(z_hbm.at[pl.ds(0, _DNB * _DCH)], bigpay)
    plsc.subcore_barrier()

    def idx_descs(i, q):
        return (
            pltpu.make_async_copy(
                col_hbm.at[pl.ds(ebase + i * _DCH, _DCH)], colb[q], isem[q]),
            pltpu.make_async_copy(
                w4_hbm.at[pl.ds(ebase + i * _DCH, _DCH)], wbuf[q], isem[q]),
        )

    def scatter_desc(u, q):
        return pltpu.make_async_copy(pays[u], acc.at[colb[q]], ssem[u])

    def copy_w(u, q):
        pay = pays[u]
        wb = wbuf[q]

        def cp(e8, carry):
            for t in range(8):
                e = e8 * 8 + t
                pay[e, pl.ds(0, _DE)] = wb[e, :]
            return carry

        lax.fori_loop(0, _DCH // 8, cp, 0)

    def step(i, u, q, fire_next_idx, guard_sw):
        q3 = (q + 3) % _DIX
        for d in idx_descs(i, q):
            d.wait()
        if guard_sw:
            @pl.when(i >= 2)
            def _():
                scatter_desc(u, q3).wait()
        else:
            scatter_desc(u, q3).wait()
        copy_w(u, q)
        scatter_desc(u, q).start(add=True)
        if fire_next_idx:
            for d in idx_descs(i + 3, q3):
                d.start()

    for i in range(3):
        for d in idx_descs(i, i):
            d.start()

    _PER = 10

    def outer(k, carry):
        i0_ = k * _PER
        for slot in range(_PER):
            step(i0_ + slot, slot % _DNB, slot % _DIX, True, guard_sw=True)
        return carry

    lax.fori_loop(0, 240 // _PER, outer, 0)
    for i in range(240, _DNCH):
        step(i, i % _DNB, i % _DIX, fire_next_idx=(i + 3 < _DNCH),
             guard_sw=False)
    for i in range(_DNCH - 2, _DNCH):
        scatter_desc(i % _DNB, i % _DIX).wait()
    plsc.subcore_barrier()
    pltpu.sync_copy(acc.at[pl.ds(s * _RPT, _RPT)],
                    out_hbm.at[c, pl.ds(s * _RPT, _RPT)])


_NBUF = 3
_IDXB = 6
_MAIN = 120


def _make_prop():
    @functools.partial(
        pl.kernel,
        out_type=jax.ShapeDtypeStruct((_NCORE, _NPAD, _HID), jnp.float32),
        mesh=_sc_mesh,
        scratch_types=[
            pltpu.VMEM_SHARED((_NPAD, _HID), jnp.float32),
            pltpu.VMEM((_EPW,), jnp.float32),
            pltpu.VMEM((_NBUF * _CH, _HID), jnp.float32),
        ] + [pltpu.VMEM((_CH,), jnp.int32) for _ in range(2 * _IDXB)]
          + [pltpu.SemaphoreType.DMA for _ in range(2 * _NBUF + _IDXB)],
    )
    def _prop(xs_hbm, row_hbm, col_hbm, wj_hbm, z_hbm, out_hbm,
              acc, wj, bigbuf,
              r0, r1, r2, r3, r4, r5, cb0, cb1, cb2, cb3, cb4, cb5,
              g0, g1, g2, s0, s1, s2, i0, i1, i2, i3, i4, i5):
        rowb = (r0, r1, r2, r3, r4, r5)
        colb = (cb0, cb1, cb2, cb3, cb4, cb5)
        bufs = tuple(bigbuf.at[pl.ds(u * _CH, _CH)] for u in range(_NBUF))
        gsem = (g0, g1, g2)
        ssem = (s0, s1, s2)
        isem = (i0, i1, i2, i3, i4, i5)
        c = lax.axis_index("c")
        s = lax.axis_index("s")
        wid = c * _NSUB + s
        ebase = wid * _EPW
        pltpu.sync_copy(wj_hbm.at[pl.ds(ebase, _EPW)], wj)
        pltpu.sync_copy(z_hbm.at[pl.ds(s * _RPT, _RPT)],
                        acc.at[pl.ds(s * _RPT, _RPT)])
        plsc.subcore_barrier()

        def idx_descs(i, q):
            return (
                pltpu.make_async_copy(
                    row_hbm.at[pl.ds(ebase + i * _CH, _CH)], rowb[q], isem[q]),
                pltpu.make_async_copy(
                    col_hbm.at[pl.ds(ebase + i * _CH, _CH)], colb[q], isem[q]),
            )

        def fire_idx(i, q):
            for d in idx_descs(i, q):
                d.start()

        def wait_idx(i, q):
            for d in idx_descs(i, q):
                d.wait()

        def gather_desc(u, q):
            return pltpu.make_async_copy(xs_hbm.at[rowb[q]], bufs[u], gsem[u])

        def scatter_desc(u, q):
            return pltpu.make_async_copy(bufs[u], acc.at[colb[q]], ssem[u])

        def scale(i, u):
            buf = bufs[u]

            def do16(t16, carry):
                wv = wj[pl.ds(i * _CH + t16 * 16, 16)]
                e0 = t16 * 16
                for t in range(16):
                    sc = wv[t]
                    for g in range(_HID // 16):
                        buf[e0 + t, pl.ds(g * 16, 16)] = (
                            buf[e0 + t, pl.ds(g * 16, 16)] * sc)
                return carry

            lax.fori_loop(0, _CH // 16, do16, 0)

        def step(i, slot, fire_next_gather, fire_next_idx, guard_sw):
            u = slot % _NBUF
            v = (slot + 2) % _NBUF
            q2 = (slot + 2) % _IDXB
            q5 = (slot + 5) % _IDXB
            qp = (slot + 5) % _IDXB
            gather_desc(u, slot).wait()
            scale(i, u)
            if guard_sw:
                @pl.when(i >= 1)
                def _():
                    scatter_desc(v, qp).wait()
            else:
                scatter_desc(v, qp).wait()
            if fire_next_gather:
                wait_idx(i + 2, q2)
                gather_desc(v, q2).start()
            scatter_desc(u, slot).start(add=True)
            if fire_next_idx:
                fire_idx(i + 5, q5)

        for i in range(5):
            fire_idx(i, i)
        wait_idx(0, 0)
        gather_desc(0, 0).start()
        wait_idx(1, 1)
        gather_desc(1, 1).start()

        def outer(k, carry):
            i0_ = k * _IDXB
            for slot in range(_IDXB):
                step(i0_ + slot, slot, True, True, guard_sw=True)
            return carry

        lax.fori_loop(0, _MAIN // _IDXB, outer, 0)
        for i in range(_MAIN, _NCHUNK):
            step(i, i % _IDXB, fire_next_gather=(i + 2 < _NCHUNK),
                 fire_next_idx=False, guard_sw=False)
        scatter_desc((_NCHUNK - 1) % _NBUF, (_NCHUNK - 1) % _IDXB).wait()
        plsc.subcore_barrier()
        pltpu.sync_copy(acc.at[pl.ds(s * _RPT, _RPT)],
                        out_hbm.at[c, pl.ds(s * _RPT, _RPT)])

    return _prop


_prop_kernel = _make_prop()


@functools.partial(
    pl.kernel,
    out_type=jax.ShapeDtypeStruct((_PICK_PAD, _HID), jnp.float32),
    mesh=_sc_mesh,
    scratch_types=[
        pltpu.VMEM((_PPW,), jnp.int32),
        pltpu.VMEM((_PPW, _HID), jnp.float32),
        pltpu.SemaphoreType.DMA,
    ],
)
def _pick_kernel(probs_hbm, pick_hbm, out_hbm, idxv, rowsv, sem):
    c = lax.axis_index("c")
    s = lax.axis_index("s")
    wid = c * _NSUB + s
    base = wid * _PPW
    pltpu.sync_copy(pick_hbm.at[pl.ds(base, _PPW)], idxv)
    for h in range(_PPW // _CH):
        pltpu.async_copy(probs_hbm.at[idxv.at[pl.ds(h * _CH, _CH)]],
                         rowsv.at[pl.ds(h * _CH, _CH)], sem).wait()
    pltpu.sync_copy(rowsv, out_hbm.at[pl.ds(base, _PPW)])


def kernel(x, edge_index, edge_attr, pickable, params):
    p = params
    row = edge_index[0].astype(jnp.int32)
    col = edge_index[1].astype(jnp.int32)
    pick = jnp.concatenate(
        [pickable.astype(jnp.int32),
         jnp.zeros((_PICK_PAD - _NPICK,), jnp.int32)])

    m1 = jnp.concatenate([p['c%d_m1_w' % j] for j in range(4)], axis=1)
    b1 = jnp.concatenate([p['c%d_m1_b' % j] for j in range(4)])[None, :]
    enc_w = p['enc1_w'] @ p['enc2_w']
    enc_b = p['enc1_b'] @ p['enc2_w'] + p['enc2_b']
    m1c = enc_w @ m1
    b1c = enc_b @ m1 + b1[0]
    m2 = jnp.zeros((4 * _HID, _DE), jnp.float32)
    for j in range(4):
        m2 = m2.at[j * _HID:(j + 1) * _HID, j].set(p['c%d_m2_w' % j][:, 0])
    b2 = jnp.full((_DE,), -1e30, jnp.float32)
    b2 = b2.at[:4].set(jnp.stack([p['c%d_m2_b' % j][0] for j in range(4)]))
    b2 = b2[None, :]

    lin_w_pad = jnp.zeros((_HID, _HID), jnp.float32)
    lin_w_pad = lin_w_pad.at[:, :_NCLS].set(p['lin_w'])
    lin_b_pad = jnp.full((_HID,), -1e30, jnp.float32)
    lin_b_pad = lin_b_pad.at[:_NCLS].set(p['lin_b'])
    lin_b_pad = lin_b_pad[None, :]

    w4, w4t = _edge_mlp(edge_attr.T, m1c, b1c, m2, b2)

    z128 = jnp.zeros((_NPAD, _HID), jnp.float32)

    degp = _deg_kernel(col, w4, z128)
    dinv, xs = _pre0(x, degp[0], degp[1], p['c0_lin_w'])

    for j in range(4):
        accs = _prop_kernel(xs, row, col, w4t[j], z128)
        if j < 3:
            xs = _mid(accs[0], accs[1], xs, dinv,
                      p['c%d_bias' % j][None, :], p['c%d_lin_w' % (j + 1)], j)
        else:
            probs = _final(accs[0], accs[1], xs, dinv,
                           p['c3_bias'][None, :], lin_w_pad, lin_b_pad)

    picked = _pick_kernel(probs, pick)
    return picked[:_NPICK, :_NCLS]


# speedup vs baseline: 25.2443x; 1.0575x over previous
import functools

import jax
import jax.numpy as jnp
from jax import lax
from jax.experimental import pallas as pl
from jax.experimental.pallas import tpu as pltpu
from jax.experimental.pallas import tpu_sc as plsc

_N = 10000
_E = 320000
_DE = 16
_HID = 128
_NCLS = 10
_NPICK = 5000

_NCORE = 2
_NSUB = 16
_NPAD = 10240
_NW = _NCORE * _NSUB
_EPW = _E // _NW
_CH = 80
_NCHUNK = _EPW // _CH
_RPT = _NPAD // _NSUB

_PICK_PAD = 5120
_PPW = _PICK_PAD // _NW

_HIGH = jax.lax.Precision.HIGHEST

_sc_mesh = plsc.VectorSubcoreMesh(core_axis_name="c", subcore_axis_name="s")


def _emlp_body(eat_ref, m1c, b1c, m2, m2b, b2, b2t, out_ref, out_t_ref):
    eat = eat_ref[...].astype(jnp.bfloat16)
    h32 = lax.dot_general(m1c[...], eat, (((0,), (0,)), ((), ())),
                          preferred_element_type=jnp.float32)
    htb = jnp.maximum(h32.astype(jnp.bfloat16) + b1c[...], jnp.bfloat16(0.0))
    z = lax.dot_general(htb, m2b[...], (((0,), (0,)), ((), ())),
                        preferred_element_type=jnp.float32) + b2[...]
    out_ref[...] = jax.nn.sigmoid(z)
    zt = lax.dot_general(m2b[...], htb, (((0,), (0,)), ((), ())),
                         preferred_element_type=jnp.float32) + b2t[...]
    out_t_ref[...] = jax.nn.sigmoid(zt)


def _edge_mlp(eat, m1c, b1c, m2, b2):
    be = 2560
    grid = _E // be
    full = lambda r, c: pl.BlockSpec((r, c), lambda i: (0, 0))
    return pl.pallas_call(
        _emlp_body,
        grid=(grid,),
        in_specs=[
            pl.BlockSpec((_DE, be), lambda i: (0, i)),
            full(_DE, 4 * _HID), full(4 * _HID, 1),
            full(4 * _HID, _DE), full(4 * _HID, _DE), full(1, _DE),
            full(_DE, 1),
        ],
        out_specs=[
            pl.BlockSpec((be, _DE), lambda i: (i, 0)),
            pl.BlockSpec((_DE, be), lambda i: (0, i)),
        ],
        out_shape=[
            jax.ShapeDtypeStruct((_E, _DE), jnp.float32),
            jax.ShapeDtypeStruct((_DE, _E), jnp.float32),
        ],
    )(eat, m1c.astype(jnp.bfloat16), b1c.reshape(4 * _HID, 1).astype(jnp.bfloat16),
      m2, m2.astype(jnp.bfloat16), b2, b2.reshape(_DE, 1))


def _pre0_body(x_ref, pa_ref, pb_ref, w_ref, dinv_ref, xs_ref):
    deg = pa_ref[0] + pb_ref[0] + 1.0
    dv = lax.rsqrt(deg)
    dinv_ref[...] = dv
    xw = jnp.dot(x_ref[...], w_ref[...], precision=_HIGH,
                 preferred_element_type=jnp.float32)
    xs_ref[...] = dv[:, 0:1] * xw


def _pre0(x, degp, w0):
    bn = 1000
    return pl.pallas_call(
        _pre0_body,
        grid=(_N // bn,),
        in_specs=[
            pl.BlockSpec((bn, _HID), lambda i: (i, 0)),
            pl.BlockSpec((1, bn, _HID), lambda i: (0, i, 0)),
            pl.BlockSpec((1, bn, _HID), lambda i: (1, i, 0)),
            pl.BlockSpec((_HID, _HID), lambda i: (0, 0)),
        ],
        out_specs=[
            pl.BlockSpec((bn, _HID), lambda i: (i, 0)),
            pl.BlockSpec((bn, _HID), lambda i: (i, 0)),
        ],
        out_shape=[
            jax.ShapeDtypeStruct((_N, _HID), jnp.float32),
            jax.ShapeDtypeStruct((_N, _HID), jnp.float32),
        ],
    )(x, degp, degp, w0)


def _mid_body(aa_ref, ab_ref, xs_ref, dinv_ref, b_ref, w_ref, out_ref, *, j):
    dvj = dinv_ref[:, j:j + 1]
    dvn = dinv_ref[:, j + 1:j + 2]
    h = dvj * (aa_ref[0] + ab_ref[0] + xs_ref[...]) + b_ref[...]
    h = jnp.maximum(h, 0.0)
    out_ref[...] = dvn * jnp.dot(h, w_ref[...], precision=_HIGH,
                                 preferred_element_type=jnp.float32)


def _mid(accs, xs, dinv, b, w, j):
    bn = 1000
    return pl.pallas_call(
        functools.partial(_mid_body, j=j),
        grid=(_N // bn,),
        in_specs=[
            pl.BlockSpec((1, bn, _HID), lambda i: (0, i, 0)),
            pl.BlockSpec((1, bn, _HID), lambda i: (1, i, 0)),
            pl.BlockSpec((bn, _HID), lambda i: (i, 0)),
            pl.BlockSpec((bn, _HID), lambda i: (i, 0)),
            pl.BlockSpec((1, _HID), lambda i: (0, 0)),
            pl.BlockSpec((_HID, _HID), lambda i: (0, 0)),
        ],
        out_specs=pl.BlockSpec((bn, _HID), lambda i: (i, 0)),
        out_shape=jax.ShapeDtypeStruct((_N, _HID), jnp.float32),
    )(accs, accs, xs, dinv, b, w)


def _final_body(aa_ref, ab_ref, xs_ref, dinv_ref, b_ref, w_ref, lb_ref, out_ref):
    dvj = dinv_ref[:, 3:4]
    h = dvj * (aa_ref[0] + ab_ref[0] + xs_ref[...]) + b_ref[...]
    logits = jnp.dot(h, w_ref[...], precision=_HIGH,
                     preferred_element_type=jnp.float32) + lb_ref[...]
    m = jnp.max(logits, axis=1, keepdims=True)
    p = jnp.exp(logits - m)
    out_ref[...] = p / jnp.sum(p, axis=1, keepdims=True)


def _final(accs, xs, dinv, b, w_pad, lb_pad):
    bn = 1000
    return pl.pallas_call(
        _final_body,
        grid=(_N // bn,),
        in_specs=[
            pl.BlockSpec((1, bn, _HID), lambda i: (0, i, 0)),
            pl.BlockSpec((1, bn, _HID), lambda i: (1, i, 0)),
            pl.BlockSpec((bn, _HID), lambda i: (i, 0)),
            pl.BlockSpec((bn, _HID), lambda i: (i, 0)),
            pl.BlockSpec((1, _HID), lambda i: (0, 0)),
            pl.BlockSpec((_HID, _HID), lambda i: (0, 0)),
            pl.BlockSpec((1, _HID), lambda i: (0, 0)),
        ],
        out_specs=pl.BlockSpec((bn, _HID), lambda i: (i, 0)),
        out_shape=jax.ShapeDtypeStruct((_N, _HID), jnp.float32),
    )(accs, accs, xs, dinv, b, w_pad, lb_pad)


_DNB = 2
_DIX = 5
_DCH = 40
_DNCH = _EPW // _DCH


@functools.partial(
    pl.kernel,
    out_type=jax.ShapeDtypeStruct((_NCORE, _NPAD, _HID), jnp.float32),
    mesh=_sc_mesh,
    scratch_types=[
        pltpu.VMEM_SHARED((_NPAD, _HID), jnp.float32),
        pltpu.VMEM((_DNB * _DCH, _HID), jnp.float32),
    ] + [pltpu.VMEM((_DCH,), jnp.int32) for _ in range(_DIX)]
      + [pltpu.VMEM((_DCH, _DE), jnp.float32) for _ in range(_DIX)]
      + [pltpu.SemaphoreType.DMA for _ in range(_DNB + _DIX)],
)
def _deg_kernel(eflat_hbm, w4_hbm, z_hbm, out_hbm, acc, bigpay,
                cb0, cb1, cb2, cb3, cb4,
                wb0, wb1, wb2, wb3, wb4,
                s0, s1, i0, i1, i2, i3, i4):
    colb = (cb0, cb1, cb2, cb3, cb4)
    wbuf = (wb0, wb1, wb2, wb3, wb4)
    pays = tuple(bigpay.at[pl.ds(u * _DCH, _DCH)] for u in range(_DNB))
    ssem = (s0, s1)
    isem = (i0, i1, i2, i3, i4)
    c = lax.axis_index("c")
    s = lax.axis_index("s")
    wid = c * _NSUB + s
    ebase = wid * _EPW
    pltpu.sync_copy(z_hbm.at[pl.ds(s * _RPT, _RPT)],
                    acc.at[pl.ds(s * _RPT, _RPT)])
    pltpu.sync_copy(z_hbm.at[pl.ds(0, _DNB * _DCH)], bigpay)
    plsc.subcore_barrier()

    def idx_descs(i, q):
        return (
            pltpu.make_async_copy(
                eflat_hbm.at[pl.ds(_E + ebase + i * _DCH, _DCH)], colb[q],
                isem[q]),
            pltpu.make_async_copy(
                w4_hbm.at[pl.ds(ebase + i * _DCH, _DCH)], wbuf[q], isem[q]),
        )

    def scatter_desc(u, q):
        return pltpu.make_async_copy(pays[u], acc.at[colb[q]], ssem[u])

    def copy_w(u, q):
        pay = pays[u]
        wb = wbuf[q]

        def cp(e8, carry):
            for t in range(8):
                e = e8 * 8 + t
                pay[e, pl.ds(0, _DE)] = wb[e, :]
            return carry

        lax.fori_loop(0, _DCH // 8, cp, 0)

    def step(i, u, q, fire_next_idx, guard_sw):
        q3 = (q + 3) % _DIX
        for d in idx_descs(i, q):
            d.wait()
        if guard_sw:
            @pl.when(i >= 2)
            def _():
                scatter_desc(u, q3).wait()
        else:
            scatter_desc(u, q3).wait()
        copy_w(u, q)
        scatter_desc(u, q).start(add=True)
        if fire_next_idx:
            for d in idx_descs(i + 3, q3):
                d.start()

    for i in range(3):
        for d in idx_descs(i, i):
            d.start()

    _PER = 10

    def outer(k, carry):
        i0_ = k * _PER
        for slot in range(_PER):
            step(i0_ + slot, slot % _DNB, slot % _DIX, True, guard_sw=True)
        return carry

    lax.fori_loop(0, 240 // _PER, outer, 0)
    for i in range(240, _DNCH):
        step(i, i % _DNB, i % _DIX, fire_next_idx=(i + 3 < _DNCH),
             guard_sw=False)
    for i in range(_DNCH - 2, _DNCH):
        scatter_desc(i % _DNB, i % _DIX).wait()
    plsc.subcore_barrier()
    pltpu.sync_copy(acc.at[pl.ds(s * _RPT, _RPT)],
                    out_hbm.at[c, pl.ds(s * _RPT, _RPT)])


_NBUF = 3
_IDXB = 6
_MAIN = 120


def _make_prop(j):
    @functools.partial(
        pl.kernel,
        out_type=jax.ShapeDtypeStruct((_NCORE, _NPAD, _HID), jnp.float32),
        mesh=_sc_mesh,
        scratch_types=[
            pltpu.VMEM_SHARED((_NPAD, _HID), jnp.float32),
            pltpu.VMEM((_EPW,), jnp.float32),
            pltpu.VMEM((_NBUF * _CH, _HID), jnp.float32),
        ] + [pltpu.VMEM((_CH,), jnp.int32) for _ in range(2 * _IDXB)]
          + [pltpu.SemaphoreType.DMA for _ in range(2 * _NBUF + _IDXB)],
    )
    def _prop(xs_hbm, eflat_hbm, wflat_hbm, z_hbm, out_hbm,
              acc, wj, bigbuf,
              r0, r1, r2, r3, r4, r5, cb0, cb1, cb2, cb3, cb4, cb5,
              g0, g1, g2, s0, s1, s2, i0, i1, i2, i3, i4, i5):
        rowb = (r0, r1, r2, r3, r4, r5)
        colb = (cb0, cb1, cb2, cb3, cb4, cb5)
        bufs = tuple(bigbuf.at[pl.ds(u * _CH, _CH)] for u in range(_NBUF))
        gsem = (g0, g1, g2)
        ssem = (s0, s1, s2)
        isem = (i0, i1, i2, i3, i4, i5)
        c = lax.axis_index("c")
        s = lax.axis_index("s")
        wid = c * _NSUB + s
        ebase = wid * _EPW
        pltpu.sync_copy(wflat_hbm.at[pl.ds(j * _E + ebase, _EPW)], wj)
        pltpu.sync_copy(z_hbm.at[pl.ds(s * _RPT, _RPT)],
                        acc.at[pl.ds(s * _RPT, _RPT)])
        plsc.subcore_barrier()

        def idx_descs(i, q):
            return (
                pltpu.make_async_copy(
                    eflat_hbm.at[pl.ds(ebase + i * _CH, _CH)], rowb[q],
                    isem[q]),
                pltpu.make_async_copy(
                    eflat_hbm.at[pl.ds(_E + ebase + i * _CH, _CH)], colb[q],
                    isem[q]),
            )

        def fire_idx(i, q):
            for d in idx_descs(i, q):
                d.start()

        def wait_idx(i, q):
            for d in idx_descs(i, q):
                d.wait()

        def gather_desc(u, q):
            return pltpu.make_async_copy(xs_hbm.at[rowb[q]], bufs[u], gsem[u])

        def scatter_desc(u, q):
            return pltpu.make_async_copy(bufs[u], acc.at[colb[q]], ssem[u])

        def scale(i, u):
            buf = bufs[u]

            def do16(t16, carry):
                wv = wj[pl.ds(i * _CH + t16 * 16, 16)]
                e0 = t16 * 16
                for t in range(16):
                    sc = wv[t]
                    for g in range(_HID // 16):
                        buf[e0 + t, pl.ds(g * 16, 16)] = (
                            buf[e0 + t, pl.ds(g * 16, 16)] * sc)
                return carry

            lax.fori_loop(0, _CH // 16, do16, 0)

        def step(i, slot, fire_next_gather, fire_next_idx, guard_sw):
            u = slot % _NBUF
            v = (slot + 2) % _NBUF
            q2 = (slot + 2) % _IDXB
            q5 = (slot + 5) % _IDXB
            qp = (slot + 5) % _IDXB
            gather_desc(u, slot).wait()
            scale(i, u)
            if guard_sw:
                @pl.when(i >= 1)
                def _():
                    scatter_desc(v, qp).wait()
            else:
                scatter_desc(v, qp).wait()
            if fire_next_gather:
                wait_idx(i + 2, q2)
                gather_desc(v, q2).start()
            scatter_desc(u, slot).start(add=True)
            if fire_next_idx:
                fire_idx(i + 5, q5)

        for i in range(5):
            fire_idx(i, i)
        wait_idx(0, 0)
        gather_desc(0, 0).start()
        wait_idx(1, 1)
        gather_desc(1, 1).start()

        def outer(k, carry):
            i0_ = k * _IDXB
            for slot in range(_IDXB):
                step(i0_ + slot, slot, True, True, guard_sw=True)
            return carry

        lax.fori_loop(0, _MAIN // _IDXB, outer, 0)
        for i in range(_MAIN, _NCHUNK):
            step(i, i % _IDXB, fire_next_gather=(i + 2 < _NCHUNK),
                 fire_next_idx=False, guard_sw=False)
        scatter_desc((_NCHUNK - 1) % _NBUF, (_NCHUNK - 1) % _IDXB).wait()
        plsc.subcore_barrier()
        pltpu.sync_copy(acc.at[pl.ds(s * _RPT, _RPT)],
                        out_hbm.at[c, pl.ds(s * _RPT, _RPT)])

    return _prop


_prop_kernels = [_make_prop(j) for j in range(4)]


@functools.partial(
    pl.kernel,
    out_type=jax.ShapeDtypeStruct((_PICK_PAD, _HID), jnp.float32),
    mesh=_sc_mesh,
    scratch_types=[
        pltpu.VMEM((_PPW,), jnp.int32),
        pltpu.VMEM((_PPW, _HID), jnp.float32),
        pltpu.SemaphoreType.DMA,
    ],
)
def _pick_kernel(probs_hbm, pick_hbm, out_hbm, idxv, rowsv, sem):
    c = lax.axis_index("c")
    s = lax.axis_index("s")
    wid = c * _NSUB + s
    base = wid * _PPW
    pltpu.sync_copy(pick_hbm.at[pl.ds(base, _PPW)], idxv)
    for h in range(_PPW // _CH):
        pltpu.async_copy(probs_hbm.at[idxv.at[pl.ds(h * _CH, _CH)]],
                         rowsv.at[pl.ds(h * _CH, _CH)], sem).wait()
    pltpu.sync_copy(rowsv, out_hbm.at[pl.ds(base, _PPW)])


def kernel(x, edge_index, edge_attr, pickable, params):
    p = params
    eflat = edge_index.astype(jnp.int32).reshape(2 * _E)
    pick = jnp.concatenate(
        [pickable.astype(jnp.int32),
         jnp.zeros((_PICK_PAD - _NPICK,), jnp.int32)])

    m1 = jnp.concatenate([p['c%d_m1_w' % j] for j in range(4)], axis=1)
    b1 = jnp.concatenate([p['c%d_m1_b' % j] for j in range(4)])[None, :]
    enc_w = p['enc1_w'] @ p['enc2_w']
    enc_b = p['enc1_b'] @ p['enc2_w'] + p['enc2_b']
    m1c = enc_w @ m1
    b1c = enc_b @ m1 + b1[0]
    m2 = jnp.zeros((4 * _HID, _DE), jnp.float32)
    for j in range(4):
        m2 = m2.at[j * _HID:(j + 1) * _HID, j].set(p['c%d_m2_w' % j][:, 0])
    b2 = jnp.full((_DE,), -1e30, jnp.float32)
    b2 = b2.at[:4].set(jnp.stack([p['c%d_m2_b' % j][0] for j in range(4)]))
    b2 = b2[None, :]

    lin_w_pad = jnp.zeros((_HID, _HID), jnp.float32)
    lin_w_pad = lin_w_pad.at[:, :_NCLS].set(p['lin_w'])
    lin_b_pad = jnp.full((_HID,), -1e30, jnp.float32)
    lin_b_pad = lin_b_pad.at[:_NCLS].set(p['lin_b'])
    lin_b_pad = lin_b_pad[None, :]

    w4, w4t = _edge_mlp(edge_attr.T, m1c, b1c, m2, b2)

    z128 = jnp.zeros((_NPAD, _HID), jnp.float32)

    wflat = w4t.reshape(_DE * _E)
    degp = _deg_kernel(eflat, w4, z128)
    dinv, xs = _pre0(x, degp, p['c0_lin_w'])

    for j in range(4):
        accs = _prop_kernels[j](xs, eflat, wflat, z128)
        if j < 3:
            xs = _mid(accs, xs, dinv,
                      p['c%d_bias' % j][None, :], p['c%d_lin_w' % (j + 1)], j)
        else:
            probs = _final(accs, xs, dinv,
                           p['c3_bias'][None, :], lin_w_pad, lin_b_pad)

    picked = _pick_kernel(probs, pick)
    return picked[:_NPICK, :_NCLS]
